# baseline (speedup 1.0000x reference)
import functools

import jax
import jax.numpy as jnp
from jax import lax
from jax.experimental import pallas as pl
from jax.experimental.pallas import tpu as pltpu
from jax.experimental.pallas import tpu_sc as plsc

N = 10000
D = 128
NBINS = 10240
NB2 = 10112
NCORES = 2
NSUB = 16
RPT2 = NB2 // NSUB
NW = NCORES * NSUB
CHUNK = 128
CPW = 80
EPW = CHUNK * CPW
E_PAD = EPW * NW
RPT = NBINS // NSUB

_sc_mesh = plsc.VectorSubcoreMesh(core_axis_name="c", subcore_axis_name="s")


@functools.partial(
    pl.kernel,
    out_type=(
        jax.ShapeDtypeStruct((NCORES, NBINS), jnp.float32),
        jax.ShapeDtypeStruct((NCORES, NBINS), jnp.float32),
    ),
    mesh=_sc_mesh,
    scratch_types=[
        pltpu.VMEM((CPW, CHUNK), jnp.int32),
        pltpu.VMEM((CPW, CHUNK), jnp.float32),
        pltpu.VMEM((CHUNK,), jnp.float32),
        pltpu.VMEM_SHARED((NBINS,), jnp.float32),
        pltpu.VMEM_SHARED((NBINS,), jnp.float32),
    ],
)
def _hist_kernel(row_hbm, lval_hbm, ones_hbm, zeros1_hbm, deg_out, loop_out,
                 ridx_v, lval_v, ones_v, deg_acc, loop_acc):
    cid = lax.axis_index("c")
    sid = lax.axis_index("s")
    wid = sid * NCORES + cid
    pltpu.sync_copy(row_hbm.at[wid], ridx_v)
    pltpu.sync_copy(lval_hbm.at[wid], lval_v)
    pltpu.sync_copy(ones_hbm, ones_v)
    sl = pl.ds(sid * RPT, RPT)
    pltpu.sync_copy(zeros1_hbm, deg_acc.at[sl])
    pltpu.sync_copy(zeros1_hbm, loop_acc.at[sl])
    plsc.subcore_barrier()

    def body(j, carry):
        pltpu.sync_copy(ones_v, deg_acc.at[ridx_v.at[j]], add=True)
        pltpu.sync_copy(lval_v.at[j], loop_acc.at[ridx_v.at[j]], add=True)
        return carry

    lax.fori_loop(0, CPW, body, 0)
    plsc.subcore_barrier()
    pltpu.sync_copy(deg_acc.at[sl], deg_out.at[cid, sl])
    pltpu.sync_copy(loop_acc.at[sl], loop_out.at[cid, sl])


K_DEPTH = 1
CPW_A = 160
CPW_B = 0
CPW_MAX = 160
TOTC = NSUB * (CPW_A + CPW_B)
TOTC_PAD = TOTC + CPW_MAX - CPW_B


K_G = 2
L_I = 4


@functools.partial(
    pl.kernel,
    out_type=jax.ShapeDtypeStruct((NB2, D), jnp.float32),
    mesh=_sc_mesh,
    scratch_types=[
        pltpu.VMEM((L_I, CHUNK), jnp.int32),
        pltpu.VMEM((L_I, CHUNK), jnp.int32),
        pltpu.VMEM((K_G, CHUNK, D), jnp.float32),
        [pltpu.SemaphoreType.DMA] * K_G,
        [pltpu.SemaphoreType.DMA] * L_I,
        [pltpu.SemaphoreType.DMA] * L_I,
        pltpu.VMEM_SHARED((NB2, D), jnp.float32),
    ],
)
def _agg_kernel(xs_hbm, row_hbm, col_hbm, zeros2_hbm, agg_out,
                ridx_v, cidx_v, bufs, gsems, risems, cisems, acc):
    cid = lax.axis_index("c")
    sid = lax.axis_index("s")
    base = sid * CPW_A
    my_cpw = jnp.where(cid == 0, CPW_A, CPW_B)
    sl = pl.ds(sid * RPT2, RPT2)

    @pl.when(my_cpw > 0)
    def _():
        pltpu.sync_copy(zeros2_hbm, acc.at[sl])
    plsc.subcore_barrier()

    def fetch_idx(j, t):
        pltpu.async_copy(row_hbm.at[base + j], ridx_v.at[t], risems[t])
        pltpu.async_copy(col_hbm.at[base + j], cidx_v.at[t], cisems[t])

    def wait_ridx(t):
        pltpu.make_async_copy(row_hbm.at[0], ridx_v.at[t], risems[t]).wait()

    def wait_cidx(t):
        pltpu.make_async_copy(col_hbm.at[0], cidx_v.at[t], cisems[t]).wait()

    def start_gather(t, b):
        pltpu.async_copy(xs_hbm.at[ridx_v.at[t]], bufs.at[b], gsems[b])

    @pl.when(my_cpw > 0)
    def _():
        for t in range(L_I):
            fetch_idx(t, t)
        for b in range(K_G):
            wait_ridx(b)
            start_gather(b, b)

    def body(g, carry):
        j0 = g * L_I
        for u in range(L_I):
            j = j0 + u
            b = u % K_G
            pltpu.make_async_copy(
                xs_hbm.at[ridx_v.at[u]], bufs.at[b], gsems[b]).wait()
            wait_cidx(u)
            pltpu.sync_copy(bufs.at[b], acc.at[cidx_v.at[u]], add=True)

            @pl.when(j + L_I < my_cpw)
            def _():
                fetch_idx(j + L_I, u)

            @pl.when(j + K_G < my_cpw)
            def _():
                t2 = (u + K_G) % L_I
                wait_ridx(t2)
                start_gather(t2, b)
        return carry

    lax.fori_loop(0, my_cpw // L_I, body, 0)
    plsc.subcore_barrier()

    @pl.when(my_cpw > 0)
    def _():
        pltpu.sync_copy(acc.at[sl], agg_out.at[sl])


_PBLK = 1280


def _prep_body(x_ref, deg_ref, loop_ref, xs_ref, sx_ref):
    outdeg = deg_ref[:, 0:1] + deg_ref[:, 1:2]
    loopsum = loop_ref[:, 0:1] + loop_ref[:, 1:2]
    degsum = outdeg + jnp.where(loopsum > 0.0, 0.0, 1.0)
    dinv = jnp.where(degsum > 0.0, 1.0 / degsum, 0.0)
    xs = x_ref[...] * dinv
    xs_ref[...] = xs
    sx_ref[...] = jnp.where(loopsum > 0.0, 0.0, xs)


_prep_call = pl.pallas_call(
    _prep_body,
    grid=(NBINS // _PBLK,),
    in_specs=[
        pl.BlockSpec((_PBLK, D), lambda i: (i, 0)),
        pl.BlockSpec((_PBLK, 2), lambda i: (i, 0)),
        pl.BlockSpec((_PBLK, 2), lambda i: (i, 0)),
    ],
    out_specs=[
        pl.BlockSpec((_PBLK, D), lambda i: (i, 0)),
        pl.BlockSpec((_PBLK, D), lambda i: (i, 0)),
    ],
    out_shape=[
        jax.ShapeDtypeStruct((NBINS, D), jnp.float32),
        jax.ShapeDtypeStruct((NBINS, D), jnp.float32),
    ],
)


_FBLK = 1000


def _dot_t(a, w):
    return lax.dot_general(a, w, (((1,), (1,)), ((), ())),
                           preferred_element_type=jnp.float32)


def _final_body(a0_ref, sx_ref, wc_ref, bc_ref, w1_ref, b1_ref,
                w2_ref, b2_ref, w3_ref, b3_ref, o_ref):
    z = a0_ref[...] + sx_ref[...]
    z = jnp.maximum(_dot_t(z, wc_ref[...]) + bc_ref[...], 0.0)
    h1 = jnp.maximum(_dot_t(z, w1_ref[...]) + b1_ref[...], 0.0)
    h2 = jnp.maximum(_dot_t(h1, w2_ref[...]) + b2_ref[...], 0.0)
    y = jnp.sum(h2 * w3_ref[...], axis=1, keepdims=True) + b3_ref[...]
    o_ref[...] = y


def _w_spec(shape):
    return pl.BlockSpec(shape, lambda i: (0, 0))


_final_call = pl.pallas_call(
    _final_body,
    grid=(N // _FBLK,),
    in_specs=[
        pl.BlockSpec((_FBLK, D), lambda i: (i, 0)),
        pl.BlockSpec((_FBLK, D), lambda i: (i, 0)),
        _w_spec((128, 128)),
        _w_spec((1, 128)),
        _w_spec((64, 128)),
        _w_spec((1, 64)),
        _w_spec((32, 64)),
        _w_spec((1, 32)),
        _w_spec((1, 32)),
        _w_spec((1, 1)),
    ],
    out_specs=pl.BlockSpec((_FBLK, 1), lambda i: (i, 0)),
    out_shape=jax.ShapeDtypeStruct((N, 1), jnp.float32),
)


def kernel(x, edge_index, W_conv, b_conv, W1, b1, W2, b2, W3, b3):
    row = edge_index[0]
    col = edge_index[1]
    e = row.shape[0]
    padv = jnp.full((E_PAD - e,), N, dtype=jnp.int32)
    row_p = jnp.concatenate([row, padv]).reshape(NW, CPW, CHUNK)
    col_p = jnp.concatenate([col, padv]).reshape(NW, CPW, CHUNK)
    lvals = (row_p == col_p).astype(jnp.float32)
    ones = jnp.ones((CHUNK,), jnp.float32)
    zeros1 = jnp.zeros((RPT,), jnp.float32)
    zeros2 = jnp.zeros((RPT2, D), jnp.float32)

    deg_out, loop_out = _hist_kernel(row_p, lvals, ones, zeros1)

    x_pad = jnp.concatenate([x, jnp.zeros((NBINS - N, D), x.dtype)])
    xs, selfx = _prep_call(x_pad, deg_out.T, loop_out.T)

    padc = jnp.full((TOTC_PAD - TOTC, CHUNK), N, dtype=jnp.int32)
    row_flat = jnp.concatenate([row_p.reshape(TOTC, CHUNK), padc])
    col_flat = jnp.concatenate([col_p.reshape(TOTC, CHUNK), padc])
    agg = _agg_kernel(xs, row_flat, col_flat, zeros2)

    out = _final_call(
        agg[:N], selfx[:N],
        W_conv, b_conv.reshape(1, D),
        W1, b1.reshape(1, 64),
        W2, b2.reshape(1, 32),
        W3, b3.reshape(1, 1),
    )
    return out


# speedup vs baseline: 1.1108x; 1.1108x over previous
import functools

import jax
import jax.numpy as jnp
from jax import lax
from jax.experimental import pallas as pl
from jax.experimental.pallas import tpu as pltpu
from jax.experimental.pallas import tpu_sc as plsc

N = 10000
D = 128
NBINS = 10240
NB2 = 10112
NCORES = 2
NSUB = 16
RPT2 = NB2 // NSUB
NW = NCORES * NSUB
CHUNK = 128
CPW = 80
EPW = CHUNK * CPW
E_PAD = EPW * NW
RPT = NBINS // NSUB

_sc_mesh = plsc.VectorSubcoreMesh(core_axis_name="c", subcore_axis_name="s")


@functools.partial(
    pl.kernel,
    out_type=(
        jax.ShapeDtypeStruct((NCORES, NBINS), jnp.float32),
        jax.ShapeDtypeStruct((NCORES, NBINS), jnp.float32),
    ),
    mesh=_sc_mesh,
    scratch_types=[
        pltpu.VMEM((CPW, CHUNK), jnp.int32),
        pltpu.VMEM((CPW, CHUNK), jnp.float32),
        pltpu.VMEM((CHUNK,), jnp.float32),
        pltpu.VMEM_SHARED((NBINS,), jnp.float32),
        pltpu.VMEM_SHARED((NBINS,), jnp.float32),
    ],
)
def _hist_kernel(row_hbm, lval_hbm, ones_hbm, zeros1_hbm, deg_out, loop_out,
                 ridx_v, lval_v, ones_v, deg_acc, loop_acc):
    cid = lax.axis_index("c")
    sid = lax.axis_index("s")
    wid = sid * NCORES + cid
    pltpu.sync_copy(row_hbm.at[wid], ridx_v)
    pltpu.sync_copy(lval_hbm.at[wid], lval_v)
    pltpu.sync_copy(ones_hbm, ones_v)
    sl = pl.ds(sid * RPT, RPT)
    pltpu.sync_copy(zeros1_hbm, deg_acc.at[sl])
    pltpu.sync_copy(zeros1_hbm, loop_acc.at[sl])
    plsc.subcore_barrier()

    def body(j, carry):
        pltpu.sync_copy(ones_v, deg_acc.at[ridx_v.at[j]], add=True)
        pltpu.sync_copy(lval_v.at[j], loop_acc.at[ridx_v.at[j]], add=True)
        return carry

    lax.fori_loop(0, CPW, body, 0)
    plsc.subcore_barrier()
    pltpu.sync_copy(deg_acc.at[sl], deg_out.at[cid, sl])
    pltpu.sync_copy(loop_acc.at[sl], loop_out.at[cid, sl])


K_DEPTH = 1
CPW_A = 128
CPW_B = 32
CPW_MAX = 128
TOTC = NSUB * (CPW_A + CPW_B)
TOTC_PAD = TOTC + CPW_MAX - CPW_B


K_G = 2
L_I = 4


@functools.partial(
    pl.kernel,
    out_type=jax.ShapeDtypeStruct((NCORES, NB2, D), jnp.float32),
    mesh=_sc_mesh,
    scratch_types=[
        pltpu.VMEM((L_I, CHUNK), jnp.int32),
        pltpu.VMEM((L_I, CHUNK), jnp.int32),
        pltpu.VMEM((K_G, CHUNK, D), jnp.float32),
        [pltpu.SemaphoreType.DMA] * K_G,
        [pltpu.SemaphoreType.DMA] * L_I,
        [pltpu.SemaphoreType.DMA] * L_I,
        pltpu.VMEM_SHARED((NB2, D), jnp.float32),
    ],
)
def _agg_kernel(xs_hbm, row_hbm, col_hbm, zeros2_hbm, agg_out,
                ridx_v, cidx_v, bufs, gsems, risems, cisems, acc):
    cid = lax.axis_index("c")
    sid = lax.axis_index("s")
    base = jnp.where(cid == 0, sid * CPW_A, NSUB * CPW_A + sid * CPW_B)
    my_cpw = jnp.where(cid == 0, CPW_A, CPW_B)
    sl = pl.ds(sid * RPT2, RPT2)
    pltpu.sync_copy(zeros2_hbm, acc.at[sl])
    plsc.subcore_barrier()

    def fetch_idx(j, t):
        pltpu.async_copy(row_hbm.at[base + j], ridx_v.at[t], risems[t])
        pltpu.async_copy(col_hbm.at[base + j], cidx_v.at[t], cisems[t])

    def wait_ridx(t):
        pltpu.make_async_copy(row_hbm.at[0], ridx_v.at[t], risems[t]).wait()

    def wait_cidx(t):
        pltpu.make_async_copy(col_hbm.at[0], cidx_v.at[t], cisems[t]).wait()

    def start_gather(t, b):
        pltpu.async_copy(xs_hbm.at[ridx_v.at[t]], bufs.at[b], gsems[b])

    for t in range(L_I):
        fetch_idx(t, t)
    for b in range(K_G):
        wait_ridx(b)
        start_gather(b, b)

    def body(g, carry):
        j0 = g * L_I
        for u in range(L_I):
            j = j0 + u
            b = u % K_G
            pltpu.make_async_copy(
                xs_hbm.at[ridx_v.at[u]], bufs.at[b], gsems[b]).wait()
            wait_cidx(u)
            pltpu.sync_copy(bufs.at[b], acc.at[cidx_v.at[u]], add=True)

            @pl.when(j + L_I < my_cpw)
            def _():
                fetch_idx(j + L_I, u)

            @pl.when(j + K_G < my_cpw)
            def _():
                t2 = (u + K_G) % L_I
                wait_ridx(t2)
                start_gather(t2, b)
        return carry

    lax.fori_loop(0, my_cpw // L_I, body, 0)
    plsc.subcore_barrier()
    pltpu.sync_copy(acc.at[sl], agg_out.at[cid, sl])


_PBLK = 1280


def _prep_body(x_ref, deg_ref, loop_ref, xs_ref, sx_ref):
    outdeg = deg_ref[:, 0:1] + deg_ref[:, 1:2]
    loopsum = loop_ref[:, 0:1] + loop_ref[:, 1:2]
    degsum = outdeg + jnp.where(loopsum > 0.0, 0.0, 1.0)
    dinv = jnp.where(degsum > 0.0, 1.0 / degsum, 0.0)
    xs = x_ref[...] * dinv
    xs_ref[...] = xs
    sx_ref[...] = jnp.where(loopsum > 0.0, 0.0, xs)


_prep_call = pl.pallas_call(
    _prep_body,
    grid=(NBINS // _PBLK,),
    in_specs=[
        pl.BlockSpec((_PBLK, D), lambda i: (i, 0)),
        pl.BlockSpec((_PBLK, 2), lambda i: (i, 0)),
        pl.BlockSpec((_PBLK, 2), lambda i: (i, 0)),
    ],
    out_specs=[
        pl.BlockSpec((_PBLK, D), lambda i: (i, 0)),
        pl.BlockSpec((_PBLK, D), lambda i: (i, 0)),
    ],
    out_shape=[
        jax.ShapeDtypeStruct((NBINS, D), jnp.float32),
        jax.ShapeDtypeStruct((NBINS, D), jnp.float32),
    ],
)


_FBLK = 1000


def _dot_t(a, w):
    return lax.dot_general(a, w, (((1,), (1,)), ((), ())),
                           preferred_element_type=jnp.float32)


def _final_body(a0_ref, a1_ref, sx_ref, wc_ref, bc_ref, w1_ref, b1_ref,
                w2_ref, b2_ref, w3_ref, b3_ref, o_ref):
    z = a0_ref[...] + a1_ref[...] + sx_ref[...]
    z = jnp.maximum(_dot_t(z, wc_ref[...]) + bc_ref[...], 0.0)
    h1 = jnp.maximum(_dot_t(z, w1_ref[...]) + b1_ref[...], 0.0)
    h2 = jnp.maximum(_dot_t(h1, w2_ref[...]) + b2_ref[...], 0.0)
    y = jnp.sum(h2 * w3_ref[...], axis=1, keepdims=True) + b3_ref[...]
    o_ref[...] = y


def _w_spec(shape):
    return pl.BlockSpec(shape, lambda i: (0, 0))


_final_call = pl.pallas_call(
    _final_body,
    grid=(N // _FBLK,),
    in_specs=[
        pl.BlockSpec((_FBLK, D), lambda i: (i, 0)),
        pl.BlockSpec((_FBLK, D), lambda i: (i, 0)),
        pl.BlockSpec((_FBLK, D), lambda i: (i, 0)),
        _w_spec((128, 128)),
        _w_spec((1, 128)),
        _w_spec((64, 128)),
        _w_spec((1, 64)),
        _w_spec((32, 64)),
        _w_spec((1, 32)),
        _w_spec((1, 32)),
        _w_spec((1, 1)),
    ],
    out_specs=pl.BlockSpec((_FBLK, 1), lambda i: (i, 0)),
    out_shape=jax.ShapeDtypeStruct((N, 1), jnp.float32),
)


def kernel(x, edge_index, W_conv, b_conv, W1, b1, W2, b2, W3, b3):
    row = edge_index[0]
    col = edge_index[1]
    e = row.shape[0]
    padv = jnp.full((E_PAD - e,), N, dtype=jnp.int32)
    row_p = jnp.concatenate([row, padv]).reshape(NW, CPW, CHUNK)
    col_p = jnp.concatenate([col, padv]).reshape(NW, CPW, CHUNK)
    lvals = (row_p == col_p).astype(jnp.float32)
    ones = jnp.ones((CHUNK,), jnp.float32)
    zeros1 = jnp.zeros((RPT,), jnp.float32)
    zeros2 = jnp.zeros((RPT2, D), jnp.float32)

    deg_out, loop_out = _hist_kernel(row_p, lvals, ones, zeros1)

    x_pad = jnp.concatenate([x, jnp.zeros((NBINS - N, D), x.dtype)])
    xs, selfx = _prep_call(x_pad, deg_out.T, loop_out.T)

    padc = jnp.full((TOTC_PAD - TOTC, CHUNK), N, dtype=jnp.int32)
    row_flat = jnp.concatenate([row_p.reshape(TOTC, CHUNK), padc])
    col_flat = jnp.concatenate([col_p.reshape(TOTC, CHUNK), padc])
    agg = _agg_kernel(xs, row_flat, col_flat, zeros2)

    out = _final_call(
        agg[0, :N], agg[1, :N], selfx[:N],
        W_conv, b_conv.reshape(1, D),
        W1, b1.reshape(1, 64),
        W2, b2.reshape(1, 32),
        W3, b3.reshape(1, 1),
    )
    return out


# speedup vs baseline: 1.1110x; 1.0002x over previous
import functools

import jax
import jax.numpy as jnp
from jax import lax
from jax.experimental import pallas as pl
from jax.experimental.pallas import tpu as pltpu
from jax.experimental.pallas import tpu_sc as plsc

N = 10000
D = 128
NBINS = 10240
NB2 = 10112
NCORES = 2
NSUB = 16
RPT2 = NB2 // NSUB
NW = NCORES * NSUB
CHUNK = 128
CPW = 80
EPW = CHUNK * CPW
E_PAD = EPW * NW
RPT = NBINS // NSUB

_sc_mesh = plsc.VectorSubcoreMesh(core_axis_name="c", subcore_axis_name="s")


@functools.partial(
    pl.kernel,
    out_type=(
        jax.ShapeDtypeStruct((NCORES, NBINS), jnp.float32),
        jax.ShapeDtypeStruct((NCORES, NBINS), jnp.float32),
    ),
    mesh=_sc_mesh,
    scratch_types=[
        pltpu.VMEM((CPW, CHUNK), jnp.int32),
        pltpu.VMEM((CPW, CHUNK), jnp.float32),
        pltpu.VMEM((CHUNK,), jnp.float32),
        pltpu.VMEM_SHARED((NBINS,), jnp.float32),
        pltpu.VMEM_SHARED((NBINS,), jnp.float32),
    ],
)
def _hist_kernel(row_hbm, lval_hbm, ones_hbm, zeros1_hbm, deg_out, loop_out,
                 ridx_v, lval_v, ones_v, deg_acc, loop_acc):
    cid = lax.axis_index("c")
    sid = lax.axis_index("s")
    wid = sid * NCORES + cid
    pltpu.sync_copy(row_hbm.at[wid], ridx_v)
    pltpu.sync_copy(lval_hbm.at[wid], lval_v)
    pltpu.sync_copy(ones_hbm, ones_v)
    sl = pl.ds(sid * RPT, RPT)
    pltpu.sync_copy(zeros1_hbm, deg_acc.at[sl])
    pltpu.sync_copy(zeros1_hbm, loop_acc.at[sl])
    plsc.subcore_barrier()

    def body(j, carry):
        pltpu.sync_copy(ones_v, deg_acc.at[ridx_v.at[j]], add=True)
        pltpu.sync_copy(lval_v.at[j], loop_acc.at[ridx_v.at[j]], add=True)
        return carry

    lax.fori_loop(0, CPW, body, 0)
    plsc.subcore_barrier()
    pltpu.sync_copy(deg_acc.at[sl], deg_out.at[cid, sl])
    pltpu.sync_copy(loop_acc.at[sl], loop_out.at[cid, sl])


CPW_A = 128
CPW_B = 32
CPW_MAX = 128
TOTC = NSUB * (CPW_A + CPW_B)
TOTC_PAD = TOTC + CPW_MAX - CPW_B


K_G = 2
L_I = 4


@functools.partial(
    pl.kernel,
    out_type=jax.ShapeDtypeStruct((NCORES, NB2, D), jnp.float32),
    mesh=_sc_mesh,
    scratch_types=[
        pltpu.VMEM((L_I, CHUNK), jnp.int32),
        pltpu.VMEM((L_I, CHUNK), jnp.int32),
        pltpu.VMEM((K_G, CHUNK, D), jnp.float32),
        [pltpu.SemaphoreType.DMA] * K_G,
        [pltpu.SemaphoreType.DMA] * L_I,
        [pltpu.SemaphoreType.DMA] * L_I,
        pltpu.VMEM_SHARED((NB2, D), jnp.float32),
    ],
)
def _agg_kernel(xs_hbm, row_hbm, col_hbm, zeros2_hbm, agg_out,
                ridx_v, cidx_v, bufs, gsems, risems, cisems, acc):
    cid = lax.axis_index("c")
    sid = lax.axis_index("s")
    base = jnp.where(cid == 0, sid * CPW_A, NSUB * CPW_A + sid * CPW_B)
    my_cpw = jnp.where(cid == 0, CPW_A, CPW_B)
    sl = pl.ds(sid * RPT2, RPT2)
    pltpu.sync_copy(zeros2_hbm, acc.at[sl])
    plsc.subcore_barrier()

    def fetch_idx(j, t):
        pltpu.async_copy(row_hbm.at[base + j], ridx_v.at[t], risems[t])
        pltpu.async_copy(col_hbm.at[base + j], cidx_v.at[t], cisems[t])

    def wait_ridx(t):
        pltpu.make_async_copy(row_hbm.at[0], ridx_v.at[t], risems[t]).wait()

    def wait_cidx(t):
        pltpu.make_async_copy(col_hbm.at[0], cidx_v.at[t], cisems[t]).wait()

    def start_gather(t, b):
        pltpu.async_copy(xs_hbm.at[ridx_v.at[t]], bufs.at[b], gsems[b])

    for t in range(L_I):
        fetch_idx(t, t)
    for b in range(K_G):
        wait_ridx(b)
        start_gather(b, b)

    def body(g, carry):
        j0 = g * L_I
        for u in range(L_I):
            j = j0 + u
            b = u % K_G
            pltpu.make_async_copy(
                xs_hbm.at[ridx_v.at[u]], bufs.at[b], gsems[b]).wait()
            wait_cidx(u)
            pltpu.sync_copy(bufs.at[b], acc.at[cidx_v.at[u]], add=True)

            @pl.when(j + L_I < my_cpw)
            def _():
                fetch_idx(j + L_I, u)

            @pl.when(j + K_G < my_cpw)
            def _():
                t2 = (u + K_G) % L_I
                wait_ridx(t2)
                start_gather(t2, b)
        return carry

    lax.fori_loop(0, my_cpw // L_I, body, 0)
    plsc.subcore_barrier()
    pltpu.sync_copy(acc.at[sl], agg_out.at[cid, sl])


_PBLK = 1280


def _prep_body(x_ref, deg_ref, loop_ref, xs_ref, sx_ref):
    outdeg = deg_ref[:, 0:1] + deg_ref[:, 1:2]
    loopsum = loop_ref[:, 0:1] + loop_ref[:, 1:2]
    degsum = outdeg + jnp.where(loopsum > 0.0, 0.0, 1.0)
    dinv = jnp.where(degsum > 0.0, 1.0 / degsum, 0.0)
    xs = x_ref[...] * dinv
    xs_ref[...] = xs
    sx_ref[...] = jnp.where(loopsum > 0.0, 0.0, xs)


_prep_call = pl.pallas_call(
    _prep_body,
    grid=(NBINS // _PBLK,),
    in_specs=[
        pl.BlockSpec((_PBLK, D), lambda i: (i, 0)),
        pl.BlockSpec((_PBLK, 2), lambda i: (i, 0)),
        pl.BlockSpec((_PBLK, 2), lambda i: (i, 0)),
    ],
    out_specs=[
        pl.BlockSpec((_PBLK, D), lambda i: (i, 0)),
        pl.BlockSpec((_PBLK, D), lambda i: (i, 0)),
    ],
    out_shape=[
        jax.ShapeDtypeStruct((NBINS, D), jnp.float32),
        jax.ShapeDtypeStruct((NBINS, D), jnp.float32),
    ],
)


_FBLK = 1000


def _dot_t(a, w):
    return lax.dot_general(a, w, (((1,), (1,)), ((), ())),
                           preferred_element_type=jnp.float32)


def _final_body(a0_ref, a1_ref, sx_ref, wc_ref, bc_ref, w1_ref, b1_ref,
                w2_ref, b2_ref, w3_ref, b3_ref, o_ref):
    z = a0_ref[...] + a1_ref[...] + sx_ref[...]
    z = jnp.maximum(_dot_t(z, wc_ref[...]) + bc_ref[...], 0.0)
    h1 = jnp.maximum(_dot_t(z, w1_ref[...]) + b1_ref[...], 0.0)
    h2 = jnp.maximum(_dot_t(h1, w2_ref[...]) + b2_ref[...], 0.0)
    y = jnp.sum(h2 * w3_ref[...], axis=1, keepdims=True) + b3_ref[...]
    o_ref[...] = y


def _w_spec(shape):
    return pl.BlockSpec(shape, lambda i: (0, 0))


_final_call = pl.pallas_call(
    _final_body,
    grid=(N // _FBLK,),
    in_specs=[
        pl.BlockSpec((_FBLK, D), lambda i: (i, 0)),
        pl.BlockSpec((_FBLK, D), lambda i: (i, 0)),
        pl.BlockSpec((_FBLK, D), lambda i: (i, 0)),
        _w_spec((128, 128)),
        _w_spec((1, 128)),
        _w_spec((64, 128)),
        _w_spec((1, 64)),
        _w_spec((32, 64)),
        _w_spec((1, 32)),
        _w_spec((1, 32)),
        _w_spec((1, 1)),
    ],
    out_specs=pl.BlockSpec((_FBLK, 1), lambda i: (i, 0)),
    out_shape=jax.ShapeDtypeStruct((N, 1), jnp.float32),
)


def kernel(x, edge_index, W_conv, b_conv, W1, b1, W2, b2, W3, b3):
    row = edge_index[0]
    col = edge_index[1]
    e = row.shape[0]
    padv = jnp.full((E_PAD - e,), N, dtype=jnp.int32)
    row_p = jnp.concatenate([row, padv]).reshape(NW, CPW, CHUNK)
    col_p = jnp.concatenate([col, padv]).reshape(NW, CPW, CHUNK)
    lvals = (row_p == col_p).astype(jnp.float32)
    ones = jnp.ones((CHUNK,), jnp.float32)
    zeros1 = jnp.zeros((RPT,), jnp.float32)
    zeros2 = jnp.zeros((RPT2, D), jnp.float32)

    deg_out, loop_out = _hist_kernel(row_p, lvals, ones, zeros1)

    x_pad = jnp.concatenate([x, jnp.zeros((NBINS - N, D), x.dtype)])
    xs, selfx = _prep_call(x_pad, deg_out.T, loop_out.T)

    padc = jnp.full((TOTC_PAD - TOTC, CHUNK), N, dtype=jnp.int32)
    row_flat = jnp.concatenate([row_p.reshape(TOTC, CHUNK), padc])
    col_flat = jnp.concatenate([col_p.reshape(TOTC, CHUNK), padc])
    agg = _agg_kernel(xs, row_flat, col_flat, zeros2)

    out = _final_call(
        agg[0, :N], agg[1, :N], selfx[:N],
        W_conv, b_conv.reshape(1, D),
        W1, b1.reshape(1, 64),
        W2, b2.reshape(1, 32),
        W3, b3.reshape(1, 1),
    )
    return out
